# baseline (speedup 1.0000x reference)
import functools

import jax
import jax.numpy as jnp
from jax import lax
from jax.experimental import pallas as pl
from jax.experimental.pallas import tpu as pltpu
from jax.experimental.pallas import tpu_sc as plsc

N = 10000
E = 320000
W = 16
KW = 64
DEPTH = 2

NC = 2
NS = 16
NWK = NC * NS
PER_W = E // NWK
CH = 2000
NCH = PER_W // CH
CHP = CH * W // 128
ROWS_PER_TILE = 1000

EP = E * W // 128
ET = 6400
ETP = ET * W // 128

_f32 = jnp.float32


@functools.cache
def _mesh():
    return plsc.VectorSubcoreMesh(core_axis_name="c", subcore_axis_name="s",
                                  num_cores=NC, num_subcores=NS)


def _sc_gather(h, src):
    @functools.partial(
        pl.kernel,
        out_type=jax.ShapeDtypeStruct((E, W), _f32),
        mesh=_mesh(),
        compiler_params=pltpu.CompilerParams(use_tc_tiling_on_sc=False),
        scratch_types=[
            pltpu.VMEM((CH,), jnp.int32),
            pltpu.VMEM((CH, W), _f32),
            pltpu.SemaphoreType.DMA,
        ],
    )
    def gk(h_hbm, src_hbm, out_hbm, idx_v, rows_v, sem):
        wid = lax.axis_index("s") * NC + lax.axis_index("c")
        base = wid * PER_W
        for ci in range(NCH):
            off = base + ci * CH
            pltpu.sync_copy(src_hbm.at[pl.ds(off, CH)], idx_v)
            pltpu.async_copy(h_hbm.at[idx_v], rows_v, sem).wait()
            pltpu.sync_copy(rows_v, out_hbm.at[pl.ds(off, CH)])

    return gk(h, src)


def _sc_scatter(vals, dst, zeros):
    @functools.partial(
        pl.kernel,
        out_type=jax.ShapeDtypeStruct((NC, N, W), _f32),
        mesh=_mesh(),
        compiler_params=pltpu.CompilerParams(use_tc_tiling_on_sc=False),
        scratch_types=[
            pltpu.VMEM((CH,), jnp.int32),
            pltpu.VMEM((CH, W), _f32),
            pltpu.VMEM_SHARED((N, W), _f32),
            pltpu.SemaphoreType.DMA,
        ],
    )
    def sk(vals_hbm, dst_hbm, z_hbm, out_hbm, idx_v, vals_v, table_s, sem):
        cid = lax.axis_index("c")
        sid = lax.axis_index("s")
        wid = sid * NC + cid

        @pl.when(sid < 10)
        def _():
            r0 = sid * ROWS_PER_TILE
            pltpu.sync_copy(z_hbm.at[pl.ds(r0, ROWS_PER_TILE)],
                            table_s.at[pl.ds(r0, ROWS_PER_TILE)])

        plsc.subcore_barrier()
        base = wid * PER_W
        for ci in range(NCH):
            off = base + ci * CH
            pltpu.sync_copy(dst_hbm.at[pl.ds(off, CH)], idx_v)
            pltpu.sync_copy(vals_hbm.at[pl.ds(off, CH)], vals_v)
            pltpu.sync_copy(vals_v, table_s.at[idx_v], add=True)
        plsc.subcore_barrier()

        @pl.when(sid < 10)
        def _():
            r0 = sid * ROWS_PER_TILE
            pltpu.sync_copy(table_s.at[pl.ds(r0, ROWS_PER_TILE)],
                            out_hbm.at[cid, pl.ds(r0, ROWS_PER_TILE)])

    return sk(vals, dst, zeros)


def _sc_count(dst, zeros, ones_small):
    @functools.partial(
        pl.kernel,
        out_type=jax.ShapeDtypeStruct((NC, N, W), _f32),
        mesh=_mesh(),
        compiler_params=pltpu.CompilerParams(use_tc_tiling_on_sc=False),
        scratch_types=[
            pltpu.VMEM((CH,), jnp.int32),
            pltpu.VMEM((CH, W), _f32),
            pltpu.VMEM_SHARED((N, W), _f32),
            pltpu.SemaphoreType.DMA,
        ],
    )
    def ck(dst_hbm, z_hbm, ones_hbm, out_hbm, idx_v, ones_v, table_s, sem):
        cid = lax.axis_index("c")
        sid = lax.axis_index("s")
        wid = sid * NC + cid

        pltpu.sync_copy(ones_hbm, ones_v)

        @pl.when(sid < 10)
        def _():
            r0 = sid * ROWS_PER_TILE
            pltpu.sync_copy(z_hbm.at[pl.ds(r0, ROWS_PER_TILE)],
                            table_s.at[pl.ds(r0, ROWS_PER_TILE)])

        plsc.subcore_barrier()
        base = wid * PER_W
        for ci in range(NCH):
            off = base + ci * CH
            pltpu.sync_copy(dst_hbm.at[pl.ds(off, CH)], idx_v)
            pltpu.sync_copy(ones_v, table_s.at[idx_v], add=True)
        plsc.subcore_barrier()

        @pl.when(sid < 10)
        def _():
            r0 = sid * ROWS_PER_TILE
            pltpu.sync_copy(table_s.at[pl.ds(r0, ROWS_PER_TILE)],
                            out_hbm.at[cid, pl.ds(r0, ROWS_PER_TILE)])

    return ck(dst, zeros, ones_small)


def _tc_msg(ea, xj, kW1, kb1, kW2, kb2, kW3, kb3):
    i_idx = jnp.arange(W * W, dtype=jnp.int32) // W
    o_idx = jnp.arange(W * W, dtype=jnp.int32) % W
    S = (i_idx[None, :] == jnp.arange(W, dtype=jnp.int32)[:, None]
         ).astype(_f32)
    U = (o_idx[:, None] == jnp.arange(W, dtype=jnp.int32)[None, :]
         ).astype(_f32)
    eye8 = jnp.eye(8, dtype=_f32)
    bd1 = jnp.kron(eye8, kW1)
    bd2 = jnp.kron(eye8, kW2)
    bd3 = jnp.kron(eye8, kW3)
    bds = jnp.kron(eye8, S)
    bdu = jnp.kron(eye8, U)
    b1p = jnp.tile(kb1, 8).reshape(1, 8 * KW)
    b2p = jnp.tile(kb2, 8).reshape(1, 8 * KW)
    b3p = jnp.tile(kb3, 8).reshape(1, 8 * W * W)

    eap = ea.reshape(E // 8, 32)
    ETR = ET // 8

    def body(ea_ref, xj_ref, w1, b1, w2, b2, w3, b3, s_ref, u_ref, out_ref):
        h1 = jnp.maximum(
            jnp.dot(ea_ref[...], w1[...], preferred_element_type=_f32) + b1[...], 0.0)
        h2 = jnp.maximum(
            jnp.dot(h1.astype(jnp.bfloat16), w2[...],
                    preferred_element_type=_f32) + b2[...], 0.0)
        w3v = jnp.dot(h2.astype(jnp.bfloat16), w3[...],
                      preferred_element_type=_f32) + b3[...]
        x16 = jnp.dot(xj_ref[...].astype(jnp.bfloat16), s_ref[...],
                      preferred_element_type=_f32)
        out_ref[...] = jnp.dot((x16 * w3v).astype(jnp.bfloat16), u_ref[...],
                               preferred_element_type=_f32)

    return pl.pallas_call(
        body,
        grid=(E // ET,),
        in_specs=[
            pl.BlockSpec((ETR, 32), lambda i: (i, 0)),
            pl.BlockSpec((ETR, 128), lambda i: (i, 0)),
            pl.BlockSpec((32, 8 * KW), lambda i: (0, 0)),
            pl.BlockSpec((1, 8 * KW), lambda i: (0, 0)),
            pl.BlockSpec((8 * KW, 8 * KW), lambda i: (0, 0)),
            pl.BlockSpec((1, 8 * KW), lambda i: (0, 0)),
            pl.BlockSpec((8 * KW, 8 * W * W), lambda i: (0, 0)),
            pl.BlockSpec((1, 8 * W * W), lambda i: (0, 0)),
            pl.BlockSpec((128, 8 * W * W), lambda i: (0, 0)),
            pl.BlockSpec((8 * W * W, 128), lambda i: (0, 0)),
        ],
        out_specs=pl.BlockSpec((ETR, 128), lambda i: (i, 0)),
        out_shape=jax.ShapeDtypeStruct((EP, 128), _f32),
    )(eap, xj, bd1, b1p, bd2.astype(jnp.bfloat16), b2p,
      bd3.astype(jnp.bfloat16), b3p, bds.astype(jnp.bfloat16),
      bdu.astype(jnp.bfloat16))


def _tc_init(x, W_in, b_in):
    def body(x_ref, w_ref, b_ref, out_ref):
        out_ref[...] = x_ref[...] * w_ref[...] + b_ref[...]

    return pl.pallas_call(
        body,
        out_shape=jax.ShapeDtypeStruct((N, W), _f32),
    )(x, W_in, b_in.reshape(1, W))


def _tc_update(h, parts, cnts):
    def body(h_ref, p_ref, c_ref, out_ref):
        s = p_ref[0] + p_ref[1]
        c = jnp.maximum(c_ref[0] + c_ref[1], 1.0)
        out_ref[...] = jnp.maximum(h_ref[...] + s / c, 0.0)

    return pl.pallas_call(
        body,
        out_shape=jax.ShapeDtypeStruct((N, W), _f32),
    )(h, parts, cnts)


def _tc_final(h, W_out1, b_out1, W_out2, b_out2):
    def body(h_ref, w1, b1, w2, b2, out_ref):
        t = jnp.maximum(
            jnp.dot(h_ref[...], w1[...], preferred_element_type=_f32) + b1[...], 0.0)
        out_ref[...] = jnp.dot(t, w2[...], preferred_element_type=_f32) + b2[...]

    return pl.pallas_call(
        body,
        out_shape=jax.ShapeDtypeStruct((N, 1), _f32),
    )(h, W_out1, b_out1.reshape(1, KW), W_out2, b_out2.reshape(1, 1))


def kernel(x, edge_index, edge_attr, W_in, b_in, kW1, kb1, kW2, kb2, kW3,
           kb3, W_out1, b_out1, W_out2, b_out2):
    src = edge_index[0]
    dst = edge_index[1]
    zeros = jnp.zeros((N, W), _f32)
    ones_small = jnp.ones((CH, W), _f32)

    h = _tc_init(x, W_in, b_in)
    cnt_parts = _sc_count(dst, zeros, ones_small)
    for _ in range(DEPTH):
        xj = _sc_gather(h, src).reshape(EP, 128)
        msg = _tc_msg(edge_attr, xj, kW1, kb1, kW2, kb2, kW3, kb3)
        parts = _sc_scatter(msg.reshape(E, W), dst, zeros)
        h = _tc_update(h, parts, cnt_parts)
    return _tc_final(h, W_out1, b_out1, W_out2, b_out2)


# speedup vs baseline: 1.0014x; 1.0014x over previous
import functools

import jax
import jax.numpy as jnp
from jax import lax
from jax.experimental import pallas as pl
from jax.experimental.pallas import tpu as pltpu
from jax.experimental.pallas import tpu_sc as plsc

N = 10000
E = 320000
W = 16
KW = 64
DEPTH = 2

NC = 2
NS = 16
NWK = NC * NS
PER_W = E // NWK
CH = 2000
NCH = PER_W // CH
CHP = CH * W // 128
ROWS_PER_TILE = 1000

EP = E * W // 128
ET = 6400
ETP = ET * W // 128

_f32 = jnp.float32


@functools.cache
def _mesh():
    return plsc.VectorSubcoreMesh(core_axis_name="c", subcore_axis_name="s",
                                  num_cores=NC, num_subcores=NS)


def _sc_gather(h, src):
    @functools.partial(
        pl.kernel,
        out_type=jax.ShapeDtypeStruct((E, W), _f32),
        mesh=_mesh(),
        compiler_params=pltpu.CompilerParams(use_tc_tiling_on_sc=False),
        scratch_types=[
            pltpu.VMEM((CH,), jnp.int32),
            pltpu.VMEM((CH, W), _f32),
            pltpu.SemaphoreType.DMA,
        ],
    )
    def gk(h_hbm, src_hbm, out_hbm, idx_v, rows_v, sem):
        wid = lax.axis_index("s") * NC + lax.axis_index("c")
        base = wid * PER_W
        for ci in range(NCH):
            off = base + ci * CH
            pltpu.sync_copy(src_hbm.at[pl.ds(off, CH)], idx_v)
            pltpu.async_copy(h_hbm.at[idx_v], rows_v, sem).wait()
            pltpu.sync_copy(rows_v, out_hbm.at[pl.ds(off, CH)])

    return gk(h, src)


def _sc_scatter(vals, dst, zeros):
    @functools.partial(
        pl.kernel,
        out_type=jax.ShapeDtypeStruct((NC, N, W), _f32),
        mesh=_mesh(),
        compiler_params=pltpu.CompilerParams(use_tc_tiling_on_sc=False),
        scratch_types=[
            pltpu.VMEM((CH,), jnp.int32),
            pltpu.VMEM((CH, W), _f32),
            pltpu.VMEM_SHARED((N, W), _f32),
            pltpu.SemaphoreType.DMA,
        ],
    )
    def sk(vals_hbm, dst_hbm, z_hbm, out_hbm, idx_v, vals_v, table_s, sem):
        cid = lax.axis_index("c")
        sid = lax.axis_index("s")
        wid = sid * NC + cid

        @pl.when(sid < 10)
        def _():
            r0 = sid * ROWS_PER_TILE
            pltpu.sync_copy(z_hbm.at[pl.ds(r0, ROWS_PER_TILE)],
                            table_s.at[pl.ds(r0, ROWS_PER_TILE)])

        plsc.subcore_barrier()
        base = wid * PER_W
        for ci in range(NCH):
            off = base + ci * CH
            pltpu.sync_copy(dst_hbm.at[pl.ds(off, CH)], idx_v)
            pltpu.sync_copy(vals_hbm.at[pl.ds(off, CH)], vals_v)
            pltpu.sync_copy(vals_v, table_s.at[idx_v], add=True)
        plsc.subcore_barrier()

        @pl.when(sid < 10)
        def _():
            r0 = sid * ROWS_PER_TILE
            pltpu.sync_copy(table_s.at[pl.ds(r0, ROWS_PER_TILE)],
                            out_hbm.at[cid, pl.ds(r0, ROWS_PER_TILE)])

    return sk(vals, dst, zeros)


def _sc_count(dst, zeros, ones_small):
    @functools.partial(
        pl.kernel,
        out_type=jax.ShapeDtypeStruct((NC, N, W), _f32),
        mesh=_mesh(),
        compiler_params=pltpu.CompilerParams(use_tc_tiling_on_sc=False),
        scratch_types=[
            pltpu.VMEM((CH,), jnp.int32),
            pltpu.VMEM((CH, W), _f32),
            pltpu.VMEM_SHARED((N, W), _f32),
            pltpu.SemaphoreType.DMA,
        ],
    )
    def ck(dst_hbm, z_hbm, ones_hbm, out_hbm, idx_v, ones_v, table_s, sem):
        cid = lax.axis_index("c")
        sid = lax.axis_index("s")
        wid = sid * NC + cid

        pltpu.sync_copy(ones_hbm, ones_v)

        @pl.when(sid < 10)
        def _():
            r0 = sid * ROWS_PER_TILE
            pltpu.sync_copy(z_hbm.at[pl.ds(r0, ROWS_PER_TILE)],
                            table_s.at[pl.ds(r0, ROWS_PER_TILE)])

        plsc.subcore_barrier()
        base = wid * PER_W
        for ci in range(NCH):
            off = base + ci * CH
            pltpu.sync_copy(dst_hbm.at[pl.ds(off, CH)], idx_v)
            pltpu.sync_copy(ones_v, table_s.at[idx_v], add=True)
        plsc.subcore_barrier()

        @pl.when(sid < 10)
        def _():
            r0 = sid * ROWS_PER_TILE
            pltpu.sync_copy(table_s.at[pl.ds(r0, ROWS_PER_TILE)],
                            out_hbm.at[cid, pl.ds(r0, ROWS_PER_TILE)])

    return ck(dst, zeros, ones_small)


def _tc_msg(ea, xj, kW1, kb1, kW2, kb2, kW3, kb3):
    i_idx = jnp.arange(W * W, dtype=jnp.int32) // W
    o_idx = jnp.arange(W * W, dtype=jnp.int32) % W
    S = (i_idx[None, :] == jnp.arange(W, dtype=jnp.int32)[:, None]
         ).astype(_f32)
    U = (o_idx[:, None] == jnp.arange(W, dtype=jnp.int32)[None, :]
         ).astype(_f32)
    eye8 = jnp.eye(8, dtype=_f32)
    bd1 = jnp.kron(eye8, kW1)
    bd2 = jnp.kron(eye8, kW2)
    bd3 = jnp.kron(eye8, kW3)
    bds = jnp.kron(eye8, S)
    bdu = jnp.kron(eye8, U)
    b1p = jnp.tile(kb1, 8).reshape(1, 8 * KW)
    b2p = jnp.tile(kb2, 8).reshape(1, 8 * KW)
    b3p = jnp.tile(kb3, 8).reshape(1, 8 * W * W)

    eap = ea.reshape(E // 8, 32)
    ETR = ET // 8

    def body(ea_ref, xj_ref, w1, b1, w2, b2, w3, b3, s_ref, u_ref, out_ref):
        h1 = jnp.maximum(
            jnp.dot(ea_ref[...], w1[...], preferred_element_type=_f32) + b1[...], 0.0)
        h2 = jnp.maximum(
            jnp.dot(h1.astype(jnp.bfloat16), w2[...],
                    preferred_element_type=_f32) + b2[...], 0.0)
        w3v = (jnp.dot(h2.astype(jnp.bfloat16), w3[...],
                       preferred_element_type=_f32)
               + b3[...]).astype(jnp.bfloat16)
        x16 = jnp.dot(xj_ref[...].astype(jnp.bfloat16), s_ref[...],
                      preferred_element_type=_f32).astype(jnp.bfloat16)
        out_ref[...] = jnp.dot(x16 * w3v, u_ref[...],
                               preferred_element_type=_f32)

    return pl.pallas_call(
        body,
        grid=(E // ET,),
        in_specs=[
            pl.BlockSpec((ETR, 32), lambda i: (i, 0)),
            pl.BlockSpec((ETR, 128), lambda i: (i, 0)),
            pl.BlockSpec((32, 8 * KW), lambda i: (0, 0)),
            pl.BlockSpec((1, 8 * KW), lambda i: (0, 0)),
            pl.BlockSpec((8 * KW, 8 * KW), lambda i: (0, 0)),
            pl.BlockSpec((1, 8 * KW), lambda i: (0, 0)),
            pl.BlockSpec((8 * KW, 8 * W * W), lambda i: (0, 0)),
            pl.BlockSpec((1, 8 * W * W), lambda i: (0, 0)),
            pl.BlockSpec((128, 8 * W * W), lambda i: (0, 0)),
            pl.BlockSpec((8 * W * W, 128), lambda i: (0, 0)),
        ],
        out_specs=pl.BlockSpec((ETR, 128), lambda i: (i, 0)),
        out_shape=jax.ShapeDtypeStruct((EP, 128), _f32),
    )(eap, xj, bd1, b1p, bd2.astype(jnp.bfloat16), b2p,
      bd3.astype(jnp.bfloat16), b3p,
      bds.astype(jnp.bfloat16), bdu.astype(jnp.bfloat16))


def _tc_init(x, W_in, b_in):
    def body(x_ref, w_ref, b_ref, out_ref):
        out_ref[...] = x_ref[...] * w_ref[...] + b_ref[...]

    return pl.pallas_call(
        body,
        out_shape=jax.ShapeDtypeStruct((N, W), _f32),
    )(x, W_in, b_in.reshape(1, W))


def _tc_update(h, parts, cnts):
    def body(h_ref, p_ref, c_ref, out_ref):
        s = p_ref[0] + p_ref[1]
        c = jnp.maximum(c_ref[0] + c_ref[1], 1.0)
        out_ref[...] = jnp.maximum(h_ref[...] + s / c, 0.0)

    return pl.pallas_call(
        body,
        out_shape=jax.ShapeDtypeStruct((N, W), _f32),
    )(h, parts, cnts)


def _tc_final(h, W_out1, b_out1, W_out2, b_out2):
    def body(h_ref, w1, b1, w2, b2, out_ref):
        t = jnp.maximum(
            jnp.dot(h_ref[...], w1[...], preferred_element_type=_f32) + b1[...], 0.0)
        out_ref[...] = jnp.dot(t, w2[...], preferred_element_type=_f32) + b2[...]

    return pl.pallas_call(
        body,
        out_shape=jax.ShapeDtypeStruct((N, 1), _f32),
    )(h, W_out1, b_out1.reshape(1, KW), W_out2, b_out2.reshape(1, 1))


def kernel(x, edge_index, edge_attr, W_in, b_in, kW1, kb1, kW2, kb2, kW3,
           kb3, W_out1, b_out1, W_out2, b_out2):
    src = edge_index[0]
    dst = edge_index[1]
    zeros = jnp.zeros((N, W), _f32)
    ones_small = jnp.ones((CH, W), _f32)

    h = _tc_init(x, W_in, b_in)
    cnt_parts = _sc_count(dst, zeros, ones_small)
    for _ in range(DEPTH):
        xj = _sc_gather(h, src).reshape(EP, 128)
        msg = _tc_msg(edge_attr, xj, kW1, kb1, kW2, kb2, kW3, kb3)
        parts = _sc_scatter(msg.reshape(E, W), dst, zeros)
        h = _tc_update(h, parts, cnt_parts)
    return _tc_final(h, W_out1, b_out1, W_out2, b_out2)


# speedup vs baseline: 1.1589x; 1.1573x over previous
import functools

import jax
import jax.numpy as jnp
from jax import lax
from jax.experimental import pallas as pl
from jax.experimental.pallas import tpu as pltpu
from jax.experimental.pallas import tpu_sc as plsc

N = 10000
E = 320000
W = 16
KW = 64
DEPTH = 2

NC = 2
NS = 16
NWK = NC * NS
PER_W = E // NWK
CH = 2000
NCH = PER_W // CH
CHP = CH * W // 128
ROWS_PER_TILE = 1000

EP = E * W // 128
ET = 6400
ETP = ET * W // 128

_f32 = jnp.float32


@functools.cache
def _mesh():
    return plsc.VectorSubcoreMesh(core_axis_name="c", subcore_axis_name="s",
                                  num_cores=NC, num_subcores=NS)


def _sc_gather(h, src):
    @functools.partial(
        pl.kernel,
        out_type=jax.ShapeDtypeStruct((E, W), _f32),
        mesh=_mesh(),
        compiler_params=pltpu.CompilerParams(use_tc_tiling_on_sc=False),
        scratch_types=[
            pltpu.VMEM((CH,), jnp.int32),
            pltpu.VMEM((CH, W), _f32),
            pltpu.SemaphoreType.DMA,
        ],
    )
    def gk(h_hbm, src_hbm, out_hbm, idx_v, rows_v, sem):
        wid = lax.axis_index("s") * NC + lax.axis_index("c")
        base = wid * PER_W
        for ci in range(NCH):
            off = base + ci * CH
            pltpu.sync_copy(src_hbm.at[pl.ds(off, CH)], idx_v)
            pltpu.async_copy(h_hbm.at[idx_v], rows_v, sem).wait()
            pltpu.sync_copy(rows_v, out_hbm.at[pl.ds(off, CH)])

    return gk(h, src)


def _sc_scatter(vals, dst, zeros):
    @functools.partial(
        pl.kernel,
        out_type=jax.ShapeDtypeStruct((NC, N, W), _f32),
        mesh=_mesh(),
        compiler_params=pltpu.CompilerParams(use_tc_tiling_on_sc=False),
        scratch_types=[
            pltpu.VMEM((CH,), jnp.int32),
            pltpu.VMEM((CH, W), _f32),
            pltpu.VMEM_SHARED((N, W), _f32),
            pltpu.SemaphoreType.DMA,
        ],
    )
    def sk(vals_hbm, dst_hbm, z_hbm, out_hbm, idx_v, vals_v, table_s, sem):
        cid = lax.axis_index("c")
        sid = lax.axis_index("s")
        wid = sid * NC + cid

        @pl.when(sid < 10)
        def _():
            r0 = sid * ROWS_PER_TILE
            pltpu.sync_copy(z_hbm.at[pl.ds(r0, ROWS_PER_TILE)],
                            table_s.at[pl.ds(r0, ROWS_PER_TILE)])

        plsc.subcore_barrier()
        base = wid * PER_W
        for ci in range(NCH):
            off = base + ci * CH
            pltpu.sync_copy(dst_hbm.at[pl.ds(off, CH)], idx_v)
            pltpu.sync_copy(vals_hbm.at[pl.ds(off, CH)], vals_v)
            pltpu.sync_copy(vals_v, table_s.at[idx_v], add=True)
        plsc.subcore_barrier()

        @pl.when(sid < 10)
        def _():
            r0 = sid * ROWS_PER_TILE
            pltpu.sync_copy(table_s.at[pl.ds(r0, ROWS_PER_TILE)],
                            out_hbm.at[cid, pl.ds(r0, ROWS_PER_TILE)])

    return sk(vals, dst, zeros)


def _sc_count(dst, zeros, ones_small, eaT):
    @functools.partial(
        pl.kernel,
        out_type=(jax.ShapeDtypeStruct((NC, N, W), _f32),
                  jax.ShapeDtypeStruct((E, W), _f32)),
        mesh=_mesh(),
        compiler_params=pltpu.CompilerParams(use_tc_tiling_on_sc=False,
                                             needs_layout_passes=False),
        scratch_types=[
            pltpu.VMEM((CH,), jnp.int32),
            pltpu.VMEM((CH, W), _f32),
            pltpu.VMEM((CH, W), _f32),
            pltpu.VMEM((4, CH), _f32),
            pltpu.VMEM_SHARED((N, W), _f32),
            pltpu.SemaphoreType.DMA,
        ],
    )
    def ck(dst_hbm, z_hbm, ones_hbm, eaT_hbm, out_hbm, ea16_hbm, idx_v,
           ones_v, ea_v, eaT_v, table_s, sem):
        cid = lax.axis_index("c")
        sid = lax.axis_index("s")
        wid = sid * NC + cid

        pltpu.sync_copy(ones_hbm, ones_v)
        pltpu.sync_copy(z_hbm.at[pl.ds(0, CH)], ea_v)

        @pl.when(sid < 10)
        def _():
            r0 = sid * ROWS_PER_TILE
            pltpu.sync_copy(z_hbm.at[pl.ds(r0, ROWS_PER_TILE)],
                            table_s.at[pl.ds(r0, ROWS_PER_TILE)])

        plsc.subcore_barrier()
        base = wid * PER_W
        for ci in range(NCH):
            off = base + ci * CH
            pltpu.sync_copy(dst_hbm.at[pl.ds(off, CH)], idx_v)
            pltpu.sync_copy(eaT_hbm.at[:, pl.ds(off, CH)], eaT_v)

            def interleave(g, carry):
                rows = g * 16 + jax.lax.iota(jnp.int32, 16)
                for j in range(4):
                    v = eaT_v[j, pl.ds(g * 16, 16)]
                    cols = jnp.full((16,), j, dtype=jnp.int32)
                    plsc.store_scatter(ea_v, [rows, cols], v)
                return carry

            lax.fori_loop(0, CH // 16, interleave, 0)
            pltpu.sync_copy(ea_v, ea16_hbm.at[pl.ds(off, CH)])
            pltpu.sync_copy(ones_v, table_s.at[idx_v], add=True)
        plsc.subcore_barrier()

        @pl.when(sid < 10)
        def _():
            r0 = sid * ROWS_PER_TILE
            pltpu.sync_copy(table_s.at[pl.ds(r0, ROWS_PER_TILE)],
                            out_hbm.at[cid, pl.ds(r0, ROWS_PER_TILE)])

    return ck(dst, zeros, ones_small, eaT)


def _tc_msg(eap, xj, kW1, kb1, kW2, kb2, kW3, kb3):
    i_idx = jnp.arange(W * W, dtype=jnp.int32) // W
    o_idx = jnp.arange(W * W, dtype=jnp.int32) % W
    S = (i_idx[None, :] == jnp.arange(W, dtype=jnp.int32)[:, None]
         ).astype(_f32)
    U = (o_idx[:, None] == jnp.arange(W, dtype=jnp.int32)[None, :]
         ).astype(_f32)
    eye8 = jnp.eye(8, dtype=_f32)
    kW1p = jnp.concatenate([kW1, jnp.zeros((W - 4, KW), _f32)], axis=0)
    bd1 = jnp.kron(eye8, kW1p)
    bd2 = jnp.kron(eye8, kW2)
    bd3 = jnp.kron(eye8, kW3)
    bds = jnp.kron(eye8, S)
    bdu = jnp.kron(eye8, U)
    b1p = jnp.tile(kb1, 8).reshape(1, 8 * KW)
    b2p = jnp.tile(kb2, 8).reshape(1, 8 * KW)
    b3p = jnp.tile(kb3, 8).reshape(1, 8 * W * W)

    ETR = ET // 8

    def body(ea_ref, xj_ref, w1, b1, w2, b2, w3, b3, s_ref, u_ref, out_ref):
        h1 = jnp.maximum(
            jnp.dot(ea_ref[...], w1[...], preferred_element_type=_f32) + b1[...], 0.0)
        h2 = jnp.maximum(
            jnp.dot(h1.astype(jnp.bfloat16), w2[...],
                    preferred_element_type=_f32) + b2[...], 0.0)
        w3v = (jnp.dot(h2.astype(jnp.bfloat16), w3[...],
                       preferred_element_type=_f32)
               + b3[...]).astype(jnp.bfloat16)
        x16 = jnp.dot(xj_ref[...].astype(jnp.bfloat16), s_ref[...],
                      preferred_element_type=_f32).astype(jnp.bfloat16)
        out_ref[...] = jnp.dot(x16 * w3v, u_ref[...],
                               preferred_element_type=_f32)

    return pl.pallas_call(
        body,
        grid=(E // ET,),
        in_specs=[
            pl.BlockSpec((ETR, 128), lambda i: (i, 0)),
            pl.BlockSpec((ETR, 128), lambda i: (i, 0)),
            pl.BlockSpec((128, 8 * KW), lambda i: (0, 0)),
            pl.BlockSpec((1, 8 * KW), lambda i: (0, 0)),
            pl.BlockSpec((8 * KW, 8 * KW), lambda i: (0, 0)),
            pl.BlockSpec((1, 8 * KW), lambda i: (0, 0)),
            pl.BlockSpec((8 * KW, 8 * W * W), lambda i: (0, 0)),
            pl.BlockSpec((1, 8 * W * W), lambda i: (0, 0)),
            pl.BlockSpec((128, 8 * W * W), lambda i: (0, 0)),
            pl.BlockSpec((8 * W * W, 128), lambda i: (0, 0)),
        ],
        out_specs=pl.BlockSpec((ETR, 128), lambda i: (i, 0)),
        out_shape=jax.ShapeDtypeStruct((EP, 128), _f32),
    )(eap, xj, bd1, b1p, bd2.astype(jnp.bfloat16), b2p,
      bd3.astype(jnp.bfloat16), b3p,
      bds.astype(jnp.bfloat16), bdu.astype(jnp.bfloat16))


def _tc_init(x, W_in, b_in):
    def body(x_ref, w_ref, b_ref, out_ref):
        out_ref[...] = x_ref[...] * w_ref[...] + b_ref[...]

    return pl.pallas_call(
        body,
        out_shape=jax.ShapeDtypeStruct((N, W), _f32),
    )(x, W_in, b_in.reshape(1, W))


def _tc_update(h, parts, cnts):
    def body(h_ref, p_ref, c_ref, out_ref):
        s = p_ref[0] + p_ref[1]
        c = jnp.maximum(c_ref[0] + c_ref[1], 1.0)
        out_ref[...] = jnp.maximum(h_ref[...] + s / c, 0.0)

    return pl.pallas_call(
        body,
        out_shape=jax.ShapeDtypeStruct((N, W), _f32),
    )(h, parts, cnts)


def _tc_final(h, W_out1, b_out1, W_out2, b_out2):
    def body(h_ref, w1, b1, w2, b2, out_ref):
        t = jnp.maximum(
            jnp.dot(h_ref[...], w1[...], preferred_element_type=_f32) + b1[...], 0.0)
        out_ref[...] = jnp.dot(t, w2[...], preferred_element_type=_f32) + b2[...]

    return pl.pallas_call(
        body,
        out_shape=jax.ShapeDtypeStruct((N, 1), _f32),
    )(h, W_out1, b_out1.reshape(1, KW), W_out2, b_out2.reshape(1, 1))


def kernel(x, edge_index, edge_attr, W_in, b_in, kW1, kb1, kW2, kb2, kW3,
           kb3, W_out1, b_out1, W_out2, b_out2):
    src = edge_index[0]
    dst = edge_index[1]
    zeros = jnp.zeros((N, W), _f32)
    ones_small = jnp.ones((CH, W), _f32)

    h = _tc_init(x, W_in, b_in)
    cnt_parts, ea16 = _sc_count(dst, zeros, ones_small,
                                jnp.transpose(edge_attr))
    eap = ea16.reshape(EP, 128)
    for _ in range(DEPTH):
        xj = _sc_gather(h, src).reshape(EP, 128)
        msg = _tc_msg(eap, xj, kW1, kb1, kW2, kb2, kW3, kb3)
        parts = _sc_scatter(msg.reshape(E, W), dst, zeros)
        h = _tc_update(h, parts, cnt_parts)
    return _tc_final(h, W_out1, b_out1, W_out2, b_out2)


# speedup vs baseline: 1.1858x; 1.0232x over previous
import functools

import jax
import jax.numpy as jnp
from jax import lax
from jax.experimental import pallas as pl
from jax.experimental.pallas import tpu as pltpu
from jax.experimental.pallas import tpu_sc as plsc

N = 10000
E = 320000
W = 16
KW = 64
DEPTH = 2

NC = 2
NS = 16
NWK = NC * NS
PER_W = E // NWK
CH = 2000
NCH = PER_W // CH
CHP = CH * W // 128
ROWS_PER_TILE = 1000

EP = E * W // 128
ET = 12800
ETP = ET * W // 128

_f32 = jnp.float32


@functools.cache
def _mesh():
    return plsc.VectorSubcoreMesh(core_axis_name="c", subcore_axis_name="s",
                                  num_cores=NC, num_subcores=NS)


def _sc_gather(h, src):
    @functools.partial(
        pl.kernel,
        out_type=jax.ShapeDtypeStruct((E, W), _f32),
        mesh=_mesh(),
        compiler_params=pltpu.CompilerParams(use_tc_tiling_on_sc=False),
        scratch_types=[
            pltpu.VMEM((CH,), jnp.int32),
            pltpu.VMEM((CH, W), _f32),
            pltpu.SemaphoreType.DMA,
        ],
    )
    def gk(h_hbm, src_hbm, out_hbm, idx_v, rows_v, sem):
        wid = lax.axis_index("s") * NC + lax.axis_index("c")
        base = wid * PER_W
        for ci in range(NCH):
            off = base + ci * CH
            pltpu.sync_copy(src_hbm.at[pl.ds(off, CH)], idx_v)
            pltpu.async_copy(h_hbm.at[idx_v], rows_v, sem).wait()
            pltpu.sync_copy(rows_v, out_hbm.at[pl.ds(off, CH)])

    return gk(h, src)


def _sc_scatter(vals, dst, zeros):
    @functools.partial(
        pl.kernel,
        out_type=jax.ShapeDtypeStruct((NC, N, W), _f32),
        mesh=_mesh(),
        compiler_params=pltpu.CompilerParams(use_tc_tiling_on_sc=False),
        scratch_types=[
            pltpu.VMEM((CH,), jnp.int32),
            pltpu.VMEM((CH, W), _f32),
            pltpu.VMEM_SHARED((N, W), _f32),
            pltpu.SemaphoreType.DMA,
        ],
    )
    def sk(vals_hbm, dst_hbm, z_hbm, out_hbm, idx_v, vals_v, table_s, sem):
        cid = lax.axis_index("c")
        sid = lax.axis_index("s")
        wid = sid * NC + cid

        @pl.when(sid < 10)
        def _():
            r0 = sid * ROWS_PER_TILE
            pltpu.sync_copy(z_hbm.at[pl.ds(r0, ROWS_PER_TILE)],
                            table_s.at[pl.ds(r0, ROWS_PER_TILE)])

        plsc.subcore_barrier()
        base = wid * PER_W
        for ci in range(NCH):
            off = base + ci * CH
            pltpu.sync_copy(dst_hbm.at[pl.ds(off, CH)], idx_v)
            pltpu.sync_copy(vals_hbm.at[pl.ds(off, CH)], vals_v)
            pltpu.sync_copy(vals_v, table_s.at[idx_v], add=True)
        plsc.subcore_barrier()

        @pl.when(sid < 10)
        def _():
            r0 = sid * ROWS_PER_TILE
            pltpu.sync_copy(table_s.at[pl.ds(r0, ROWS_PER_TILE)],
                            out_hbm.at[cid, pl.ds(r0, ROWS_PER_TILE)])

    return sk(vals, dst, zeros)


def _sc_count(dst, zeros, ones_small, eaT):
    @functools.partial(
        pl.kernel,
        out_type=(jax.ShapeDtypeStruct((NC, N, W), _f32),
                  jax.ShapeDtypeStruct((E, W), _f32)),
        mesh=_mesh(),
        compiler_params=pltpu.CompilerParams(use_tc_tiling_on_sc=False,
                                             needs_layout_passes=False),
        scratch_types=[
            pltpu.VMEM((CH,), jnp.int32),
            pltpu.VMEM((CH, W), _f32),
            pltpu.VMEM((CH, W), _f32),
            pltpu.VMEM((4, CH), _f32),
            pltpu.VMEM_SHARED((N, W), _f32),
            pltpu.SemaphoreType.DMA,
        ],
    )
    def ck(dst_hbm, z_hbm, ones_hbm, eaT_hbm, out_hbm, ea16_hbm, idx_v,
           ones_v, ea_v, eaT_v, table_s, sem):
        cid = lax.axis_index("c")
        sid = lax.axis_index("s")
        wid = sid * NC + cid

        pltpu.sync_copy(ones_hbm, ones_v)
        pltpu.sync_copy(z_hbm.at[pl.ds(0, CH)], ea_v)

        @pl.when(sid < 10)
        def _():
            r0 = sid * ROWS_PER_TILE
            pltpu.sync_copy(z_hbm.at[pl.ds(r0, ROWS_PER_TILE)],
                            table_s.at[pl.ds(r0, ROWS_PER_TILE)])

        plsc.subcore_barrier()
        base = wid * PER_W
        for ci in range(NCH):
            off = base + ci * CH
            pltpu.sync_copy(dst_hbm.at[pl.ds(off, CH)], idx_v)
            pltpu.sync_copy(eaT_hbm.at[:, pl.ds(off, CH)], eaT_v)

            def interleave(g, carry):
                rows = g * 16 + jax.lax.iota(jnp.int32, 16)
                for j in range(4):
                    v = eaT_v[j, pl.ds(g * 16, 16)]
                    cols = jnp.full((16,), j, dtype=jnp.int32)
                    plsc.store_scatter(ea_v, [rows, cols], v)
                return carry

            lax.fori_loop(0, CH // 16, interleave, 0)
            pltpu.sync_copy(ea_v, ea16_hbm.at[pl.ds(off, CH)])
            pltpu.sync_copy(ones_v, table_s.at[idx_v], add=True)
        plsc.subcore_barrier()

        @pl.when(sid < 10)
        def _():
            r0 = sid * ROWS_PER_TILE
            pltpu.sync_copy(table_s.at[pl.ds(r0, ROWS_PER_TILE)],
                            out_hbm.at[cid, pl.ds(r0, ROWS_PER_TILE)])

    return ck(dst, zeros, ones_small, eaT)


def _tc_msg(eap, xj, kW1, kb1, kW2, kb2, kW3, kb3):
    i_idx = jnp.arange(W * W, dtype=jnp.int32) // W
    o_idx = jnp.arange(W * W, dtype=jnp.int32) % W
    S = (i_idx[None, :] == jnp.arange(W, dtype=jnp.int32)[:, None]
         ).astype(_f32)
    U = (o_idx[:, None] == jnp.arange(W, dtype=jnp.int32)[None, :]
         ).astype(_f32)
    eye8 = jnp.eye(8, dtype=_f32)
    kW1p = jnp.concatenate([kW1, jnp.zeros((W - 4, KW), _f32)], axis=0)
    bd1 = jnp.kron(eye8, kW1p)
    bd2 = jnp.kron(eye8, kW2)
    bd3 = jnp.kron(eye8, kW3)
    bds = jnp.kron(eye8, S)
    bdu = jnp.kron(eye8, U)
    b1p = jnp.tile(kb1, 8).reshape(1, 8 * KW)
    b2p = jnp.tile(kb2, 8).reshape(1, 8 * KW)
    b3p = jnp.tile(kb3, 8).reshape(1, 8 * W * W)

    ETR = ET // 8

    def body(ea_ref, xj_ref, w1, b1, w2, b2, w3, b3, s_ref, u_ref, out_ref):
        h1 = jnp.maximum(
            jnp.dot(ea_ref[...], w1[...], preferred_element_type=_f32) + b1[...], 0.0)
        h2 = jnp.maximum(
            jnp.dot(h1.astype(jnp.bfloat16), w2[...],
                    preferred_element_type=_f32) + b2[...], 0.0)
        w3v = (jnp.dot(h2.astype(jnp.bfloat16), w3[...],
                       preferred_element_type=_f32)
               + b3[...]).astype(jnp.bfloat16)
        x16 = jnp.dot(xj_ref[...].astype(jnp.bfloat16), s_ref[...],
                      preferred_element_type=_f32).astype(jnp.bfloat16)
        out_ref[...] = jnp.dot(x16 * w3v, u_ref[...],
                               preferred_element_type=_f32)

    return pl.pallas_call(
        body,
        grid=(E // ET,),
        in_specs=[
            pl.BlockSpec((ETR, 128), lambda i: (i, 0)),
            pl.BlockSpec((ETR, 128), lambda i: (i, 0)),
            pl.BlockSpec((128, 8 * KW), lambda i: (0, 0)),
            pl.BlockSpec((1, 8 * KW), lambda i: (0, 0)),
            pl.BlockSpec((8 * KW, 8 * KW), lambda i: (0, 0)),
            pl.BlockSpec((1, 8 * KW), lambda i: (0, 0)),
            pl.BlockSpec((8 * KW, 8 * W * W), lambda i: (0, 0)),
            pl.BlockSpec((1, 8 * W * W), lambda i: (0, 0)),
            pl.BlockSpec((128, 8 * W * W), lambda i: (0, 0)),
            pl.BlockSpec((8 * W * W, 128), lambda i: (0, 0)),
        ],
        out_specs=pl.BlockSpec((ETR, 128), lambda i: (i, 0)),
        out_shape=jax.ShapeDtypeStruct((EP, 128), _f32),
    )(eap, xj, bd1, b1p, bd2.astype(jnp.bfloat16), b2p,
      bd3.astype(jnp.bfloat16), b3p,
      bds.astype(jnp.bfloat16), bdu.astype(jnp.bfloat16))


def _tc_init(x, W_in, b_in):
    def body(x_ref, w_ref, b_ref, out_ref):
        out_ref[...] = x_ref[...] * w_ref[...] + b_ref[...]

    return pl.pallas_call(
        body,
        out_shape=jax.ShapeDtypeStruct((N, W), _f32),
    )(x, W_in, b_in.reshape(1, W))


def _tc_update(h, parts, cnts):
    def body(h_ref, p_ref, c_ref, out_ref):
        s = p_ref[0] + p_ref[1]
        c = jnp.maximum(c_ref[0] + c_ref[1], 1.0)
        out_ref[...] = jnp.maximum(h_ref[...] + s / c, 0.0)

    return pl.pallas_call(
        body,
        out_shape=jax.ShapeDtypeStruct((N, W), _f32),
    )(h, parts, cnts)


def _tc_final(h, W_out1, b_out1, W_out2, b_out2):
    def body(h_ref, w1, b1, w2, b2, out_ref):
        t = jnp.maximum(
            jnp.dot(h_ref[...], w1[...], preferred_element_type=_f32) + b1[...], 0.0)
        out_ref[...] = jnp.dot(t, w2[...], preferred_element_type=_f32) + b2[...]

    return pl.pallas_call(
        body,
        out_shape=jax.ShapeDtypeStruct((N, 1), _f32),
    )(h, W_out1, b_out1.reshape(1, KW), W_out2, b_out2.reshape(1, 1))


def kernel(x, edge_index, edge_attr, W_in, b_in, kW1, kb1, kW2, kb2, kW3,
           kb3, W_out1, b_out1, W_out2, b_out2):
    src = edge_index[0]
    dst = edge_index[1]
    zeros = jnp.zeros((N, W), _f32)
    ones_small = jnp.ones((CH, W), _f32)

    h = _tc_init(x, W_in, b_in)
    cnt_parts, ea16 = _sc_count(dst, zeros, ones_small,
                                jnp.transpose(edge_attr))
    eap = ea16.reshape(EP, 128)
    for _ in range(DEPTH):
        xj = _sc_gather(h, src).reshape(EP, 128)
        msg = _tc_msg(eap, xj, kW1, kb1, kW2, kb2, kW3, kb3)
        parts = _sc_scatter(msg.reshape(E, W), dst, zeros)
        h = _tc_update(h, parts, cnt_parts)
    return _tc_final(h, W_out1, b_out1, W_out2, b_out2)


# speedup vs baseline: 1.1956x; 1.0083x over previous
import functools

import jax
import jax.numpy as jnp
from jax import lax
from jax.experimental import pallas as pl
from jax.experimental.pallas import tpu as pltpu
from jax.experimental.pallas import tpu_sc as plsc

N = 10000
E = 320000
W = 16
KW = 64
DEPTH = 2

NC = 2
NS = 16
NWK = NC * NS
PER_W = E // NWK
CH = 2000
NCH = PER_W // CH
CHP = CH * W // 128
ROWS_PER_TILE = 1000

EP = E * W // 128
ET = 32000
ETP = ET * W // 128

_f32 = jnp.float32


@functools.cache
def _mesh():
    return plsc.VectorSubcoreMesh(core_axis_name="c", subcore_axis_name="s",
                                  num_cores=NC, num_subcores=NS)


def _sc_gather(h, src):
    @functools.partial(
        pl.kernel,
        out_type=jax.ShapeDtypeStruct((E, W), _f32),
        mesh=_mesh(),
        compiler_params=pltpu.CompilerParams(use_tc_tiling_on_sc=False),
        scratch_types=[
            pltpu.VMEM((CH,), jnp.int32),
            pltpu.VMEM((CH, W), _f32),
            pltpu.SemaphoreType.DMA,
        ],
    )
    def gk(h_hbm, src_hbm, out_hbm, idx_v, rows_v, sem):
        wid = lax.axis_index("s") * NC + lax.axis_index("c")
        base = wid * PER_W
        for ci in range(NCH):
            off = base + ci * CH
            pltpu.sync_copy(src_hbm.at[pl.ds(off, CH)], idx_v)
            pltpu.async_copy(h_hbm.at[idx_v], rows_v, sem).wait()
            pltpu.sync_copy(rows_v, out_hbm.at[pl.ds(off, CH)])

    return gk(h, src)


def _sc_scatter(vals, dst, zeros):
    @functools.partial(
        pl.kernel,
        out_type=jax.ShapeDtypeStruct((NC, N, W), _f32),
        mesh=_mesh(),
        compiler_params=pltpu.CompilerParams(use_tc_tiling_on_sc=False),
        scratch_types=[
            pltpu.VMEM((CH,), jnp.int32),
            pltpu.VMEM((CH, W), _f32),
            pltpu.VMEM_SHARED((N, W), _f32),
            pltpu.SemaphoreType.DMA,
        ],
    )
    def sk(vals_hbm, dst_hbm, z_hbm, out_hbm, idx_v, vals_v, table_s, sem):
        cid = lax.axis_index("c")
        sid = lax.axis_index("s")
        wid = sid * NC + cid

        @pl.when(sid < 10)
        def _():
            r0 = sid * ROWS_PER_TILE
            pltpu.sync_copy(z_hbm.at[pl.ds(r0, ROWS_PER_TILE)],
                            table_s.at[pl.ds(r0, ROWS_PER_TILE)])

        plsc.subcore_barrier()
        base = wid * PER_W
        for ci in range(NCH):
            off = base + ci * CH
            pltpu.sync_copy(dst_hbm.at[pl.ds(off, CH)], idx_v)
            pltpu.sync_copy(vals_hbm.at[pl.ds(off, CH)], vals_v)
            pltpu.sync_copy(vals_v, table_s.at[idx_v], add=True)
        plsc.subcore_barrier()

        @pl.when(sid < 10)
        def _():
            r0 = sid * ROWS_PER_TILE
            pltpu.sync_copy(table_s.at[pl.ds(r0, ROWS_PER_TILE)],
                            out_hbm.at[cid, pl.ds(r0, ROWS_PER_TILE)])

    return sk(vals, dst, zeros)


def _sc_count(dst, zeros, ones_small, eaT):
    @functools.partial(
        pl.kernel,
        out_type=(jax.ShapeDtypeStruct((NC, N, W), _f32),
                  jax.ShapeDtypeStruct((E, W), _f32)),
        mesh=_mesh(),
        compiler_params=pltpu.CompilerParams(use_tc_tiling_on_sc=False,
                                             needs_layout_passes=False),
        scratch_types=[
            pltpu.VMEM((CH,), jnp.int32),
            pltpu.VMEM((CH, W), _f32),
            pltpu.VMEM((CH, W), _f32),
            pltpu.VMEM((4, CH), _f32),
            pltpu.VMEM_SHARED((N, W), _f32),
            pltpu.SemaphoreType.DMA,
        ],
    )
    def ck(dst_hbm, z_hbm, ones_hbm, eaT_hbm, out_hbm, ea16_hbm, idx_v,
           ones_v, ea_v, eaT_v, table_s, sem):
        cid = lax.axis_index("c")
        sid = lax.axis_index("s")
        wid = sid * NC + cid

        pltpu.sync_copy(ones_hbm, ones_v)
        pltpu.sync_copy(z_hbm.at[pl.ds(0, CH)], ea_v)

        @pl.when(sid < 10)
        def _():
            r0 = sid * ROWS_PER_TILE
            pltpu.sync_copy(z_hbm.at[pl.ds(r0, ROWS_PER_TILE)],
                            table_s.at[pl.ds(r0, ROWS_PER_TILE)])

        plsc.subcore_barrier()
        base = wid * PER_W
        for ci in range(NCH):
            off = base + ci * CH
            pltpu.sync_copy(dst_hbm.at[pl.ds(off, CH)], idx_v)
            pltpu.sync_copy(eaT_hbm.at[:, pl.ds(off, CH)], eaT_v)

            def interleave(g, carry):
                rows = g * 16 + jax.lax.iota(jnp.int32, 16)
                for j in range(4):
                    v = eaT_v[j, pl.ds(g * 16, 16)]
                    cols = jnp.full((16,), j, dtype=jnp.int32)
                    plsc.store_scatter(ea_v, [rows, cols], v)
                return carry

            lax.fori_loop(0, CH // 16, interleave, 0)
            pltpu.sync_copy(ea_v, ea16_hbm.at[pl.ds(off, CH)])
            pltpu.sync_copy(ones_v, table_s.at[idx_v], add=True)
        plsc.subcore_barrier()

        @pl.when(sid < 10)
        def _():
            r0 = sid * ROWS_PER_TILE
            pltpu.sync_copy(table_s.at[pl.ds(r0, ROWS_PER_TILE)],
                            out_hbm.at[cid, pl.ds(r0, ROWS_PER_TILE)])

    return ck(dst, zeros, ones_small, eaT)


def _tc_msg(eap, xj, kW1, kb1, kW2, kb2, kW3, kb3):
    i_idx = jnp.arange(W * W, dtype=jnp.int32) // W
    o_idx = jnp.arange(W * W, dtype=jnp.int32) % W
    S = (i_idx[None, :] == jnp.arange(W, dtype=jnp.int32)[:, None]
         ).astype(_f32)
    U = (o_idx[:, None] == jnp.arange(W, dtype=jnp.int32)[None, :]
         ).astype(_f32)
    eye8 = jnp.eye(8, dtype=_f32)
    kW1p = jnp.concatenate([kW1, jnp.zeros((W - 4, KW), _f32)], axis=0)
    bd1 = jnp.kron(eye8, kW1p)
    bd2 = jnp.kron(eye8, kW2)
    bd3 = jnp.kron(eye8, kW3)
    bds = jnp.kron(eye8, S)
    bdu = jnp.kron(eye8, U)
    b1p = jnp.tile(kb1, 8).reshape(1, 8 * KW)
    b2p = jnp.tile(kb2, 8).reshape(1, 8 * KW)
    b3p = jnp.tile(kb3, 8).reshape(1, 8 * W * W)

    ETR = ET // 8

    def body(ea_ref, xj_ref, w1, b1, w2, b2, w3, b3, s_ref, u_ref, out_ref):
        h1 = jnp.maximum(
            jnp.dot(ea_ref[...], w1[...], preferred_element_type=_f32) + b1[...], 0.0)
        h2 = jnp.maximum(
            jnp.dot(h1.astype(jnp.bfloat16), w2[...],
                    preferred_element_type=_f32) + b2[...], 0.0)
        w3v = (jnp.dot(h2.astype(jnp.bfloat16), w3[...],
                       preferred_element_type=_f32)
               + b3[...]).astype(jnp.bfloat16)
        x16 = jnp.dot(xj_ref[...].astype(jnp.bfloat16), s_ref[...],
                      preferred_element_type=_f32).astype(jnp.bfloat16)
        out_ref[...] = jnp.dot(x16 * w3v, u_ref[...],
                               preferred_element_type=_f32)

    return pl.pallas_call(
        body,
        grid=(E // ET,),
        in_specs=[
            pl.BlockSpec((ETR, 128), lambda i: (i, 0)),
            pl.BlockSpec((ETR, 128), lambda i: (i, 0)),
            pl.BlockSpec((128, 8 * KW), lambda i: (0, 0)),
            pl.BlockSpec((1, 8 * KW), lambda i: (0, 0)),
            pl.BlockSpec((8 * KW, 8 * KW), lambda i: (0, 0)),
            pl.BlockSpec((1, 8 * KW), lambda i: (0, 0)),
            pl.BlockSpec((8 * KW, 8 * W * W), lambda i: (0, 0)),
            pl.BlockSpec((1, 8 * W * W), lambda i: (0, 0)),
            pl.BlockSpec((128, 8 * W * W), lambda i: (0, 0)),
            pl.BlockSpec((8 * W * W, 128), lambda i: (0, 0)),
        ],
        out_specs=pl.BlockSpec((ETR, 128), lambda i: (i, 0)),
        out_shape=jax.ShapeDtypeStruct((EP, 128), _f32),
    )(eap, xj, bd1, b1p, bd2.astype(jnp.bfloat16), b2p,
      bd3.astype(jnp.bfloat16), b3p,
      bds.astype(jnp.bfloat16), bdu.astype(jnp.bfloat16))


def _tc_init(x, W_in, b_in):
    def body(x_ref, w_ref, b_ref, out_ref):
        out_ref[...] = x_ref[...] * w_ref[...] + b_ref[...]

    return pl.pallas_call(
        body,
        out_shape=jax.ShapeDtypeStruct((N, W), _f32),
    )(x, W_in, b_in.reshape(1, W))


def _tc_update(h, parts, cnts):
    def body(h_ref, p_ref, c_ref, out_ref):
        s = p_ref[0] + p_ref[1]
        c = jnp.maximum(c_ref[0] + c_ref[1], 1.0)
        out_ref[...] = jnp.maximum(h_ref[...] + s / c, 0.0)

    return pl.pallas_call(
        body,
        out_shape=jax.ShapeDtypeStruct((N, W), _f32),
    )(h, parts, cnts)


def _tc_final(h, W_out1, b_out1, W_out2, b_out2):
    def body(h_ref, w1, b1, w2, b2, out_ref):
        t = jnp.maximum(
            jnp.dot(h_ref[...], w1[...], preferred_element_type=_f32) + b1[...], 0.0)
        out_ref[...] = jnp.dot(t, w2[...], preferred_element_type=_f32) + b2[...]

    return pl.pallas_call(
        body,
        out_shape=jax.ShapeDtypeStruct((N, 1), _f32),
    )(h, W_out1, b_out1.reshape(1, KW), W_out2, b_out2.reshape(1, 1))


def kernel(x, edge_index, edge_attr, W_in, b_in, kW1, kb1, kW2, kb2, kW3,
           kb3, W_out1, b_out1, W_out2, b_out2):
    src = edge_index[0]
    dst = edge_index[1]
    zeros = jnp.zeros((N, W), _f32)
    ones_small = jnp.ones((CH, W), _f32)

    h = _tc_init(x, W_in, b_in)
    cnt_parts, ea16 = _sc_count(dst, zeros, ones_small,
                                jnp.transpose(edge_attr))
    eap = ea16.reshape(EP, 128)
    for _ in range(DEPTH):
        xj = _sc_gather(h, src).reshape(EP, 128)
        msg = _tc_msg(eap, xj, kW1, kb1, kW2, kb2, kW3, kb3)
        parts = _sc_scatter(msg.reshape(E, W), dst, zeros)
        h = _tc_update(h, parts, cnt_parts)
    return _tc_final(h, W_out1, b_out1, W_out2, b_out2)
